# Initial kernel scaffold; baseline (speedup 1.0000x reference)
import jax
import jax.numpy as jnp
from jax.experimental import pallas as pl


def kernel(x, edge_index, W0, b0, W1, b1):
    raise NotImplementedError("write your pallas kernel here")


# speedup vs baseline: 10.6096x; 10.6096x over previous
import functools

import jax
import jax.numpy as jnp
from jax import lax
from jax.experimental import pallas as pl
from jax.experimental.pallas import tpu as pltpu
from jax.experimental.pallas import tpu_sc as plsc

N = 10000
E = 320000
D_IN = 128
D_H = 256
D_OUT = 1024

NC = 2
NS = 16
LANES = 16
NW = NC * NS

NP = 10240
EP = 327680
ROWS = EP // 128
RPS = ROWS // NS
RPW = ROWS // NW
NPS = NP // NS

_mesh = plsc.VectorSubcoreMesh(core_axis_name="c", subcore_axis_name="s")


@functools.partial(
    pl.kernel,
    out_type=jax.ShapeDtypeStruct((NC, NP, LANES), jnp.float32),
    mesh=_mesh,
    scratch_types=[
        pltpu.VMEM((RPW, 128), jnp.int32),
        pltpu.VMEM((128, LANES), jnp.float32),
        pltpu.VMEM((128, LANES), jnp.float32),
        pltpu.VMEM_SHARED((NP, LANES), jnp.float32),
    ],
)
def _deg_kernel(dst_hbm, deg_hbm, dst_v, ones_v, zeros_v, hist_sp):
    c = lax.axis_index("c")
    s = lax.axis_index("s")
    w = c * NS + s

    pltpu.sync_copy(dst_hbm.at[pl.ds(w * RPW, RPW)], dst_v)

    one = jnp.full((LANES,), 1.0, jnp.float32)
    zero = jnp.zeros((LANES,), jnp.float32)

    @pl.loop(0, 128)
    def _(i):
        ones_v[i] = one
        zeros_v[i] = zero

    @pl.loop(0, NPS // 128)
    def _(k):
        pltpu.sync_copy(zeros_v, hist_sp.at[pl.ds(s * NPS + k * 128, 128)])

    plsc.subcore_barrier()

    @pl.loop(0, RPW)
    def _(t):
        pltpu.sync_copy(ones_v, hist_sp.at[dst_v.at[t]], add=True)

    plsc.subcore_barrier()
    pltpu.sync_copy(
        hist_sp.at[pl.ds(s * NPS, NPS)],
        deg_hbm.at[c].at[pl.ds(s * NPS, NPS)],
    )


@functools.partial(
    pl.kernel,
    out_type=jax.ShapeDtypeStruct((NC * NP, D_IN), jnp.float32),
    mesh=_mesh,
    scratch_types=[
        pltpu.VMEM((RPW, 128), jnp.int32),
        pltpu.VMEM((RPW, 128), jnp.int32),
        pltpu.VMEM((128, D_IN), jnp.float32),
        pltpu.VMEM_SHARED((NP, D_IN), jnp.float32),
    ],
)
def _prop_es(src_hbm, dst_hbm, xs_hbm, out_hbm, src_v, dst_v, data_v, acc_sp):
    c = lax.axis_index("c")
    s = lax.axis_index("s")
    base = c * (ROWS // NC) + s * RPW

    pltpu.sync_copy(src_hbm.at[pl.ds(base, RPW)], src_v)
    pltpu.sync_copy(dst_hbm.at[pl.ds(base, RPW)], dst_v)

    @pl.when(c == 0)
    def _():
        pltpu.sync_copy(
            xs_hbm.at[pl.ds(s * NPS, NPS)],
            acc_sp.at[pl.ds(s * NPS, NPS)],
        )

    @pl.when(c != 0)
    def _():
        zero = jnp.zeros((LANES,), jnp.float32)

        @pl.loop(0, 128)
        def _(i):
            @pl.loop(0, D_IN, step=LANES)
            def _(j):
                data_v[i, pl.ds(j, LANES)] = zero

        @pl.loop(0, NPS // 128)
        def _(k):
            pltpu.sync_copy(data_v, acc_sp.at[pl.ds(s * NPS + k * 128, 128)])

    plsc.subcore_barrier()

    @pl.loop(0, RPW)
    def _(t):
        pltpu.sync_copy(xs_hbm.at[src_v.at[t]], data_v)
        pltpu.sync_copy(data_v, acc_sp.at[dst_v.at[t]], add=True)

    plsc.subcore_barrier()
    pltpu.sync_copy(
        acc_sp.at[pl.ds(s * NPS, NPS)],
        out_hbm.at[pl.ds(c * NP + s * NPS, NPS)],
    )


@functools.partial(
    pl.kernel,
    out_type=jax.ShapeDtypeStruct((NC * NP, D_H // 2), jnp.float32),
    mesh=_mesh,
    scratch_types=[
        pltpu.VMEM((RPS // 2, 128), jnp.int32),
        pltpu.VMEM((RPS // 2, 128), jnp.int32),
        pltpu.VMEM((128, D_H // 2), jnp.float32),
        pltpu.VMEM_SHARED((NP, D_H // 2), jnp.float32),
    ],
)
def _prop_cs(src_hbm, dst_hbm, xs_hbm, out_hbm, src_v, dst_v, data_v, acc_sp):
    c = lax.axis_index("c")
    s = lax.axis_index("s")
    off = jnp.broadcast_to((c * NP).astype(jnp.int32), (LANES,))
    half = RPS // 2

    pltpu.sync_copy(
        xs_hbm.at[pl.ds(c * NP + s * NPS, NPS)],
        acc_sp.at[pl.ds(s * NPS, NPS)],
    )
    plsc.subcore_barrier()

    for p in range(2):
        pltpu.sync_copy(src_hbm.at[pl.ds(s * RPS + p * half, half)], src_v)
        pltpu.sync_copy(dst_hbm.at[pl.ds(s * RPS + p * half, half)], dst_v)

        @pl.loop(0, half)
        def _(i):
            @pl.loop(0, 128, step=LANES)
            def _(j):
                src_v[i, pl.ds(j, LANES)] = src_v[i, pl.ds(j, LANES)] + off

        @pl.loop(0, half)
        def _(t):
            pltpu.sync_copy(xs_hbm.at[src_v.at[t]], data_v)
            pltpu.sync_copy(data_v, acc_sp.at[dst_v.at[t]], add=True)

    plsc.subcore_barrier()
    pltpu.sync_copy(
        acc_sp.at[pl.ds(s * NPS, NPS)],
        out_hbm.at[pl.ds(c * NP + s * NPS, NPS)],
    )


BN = 512


def _mm1_body(p_ref, w_ref, dinv_ref, b_ref, o_ref):
    q = p_ref[0] + p_ref[1]
    acc = jnp.dot(q, w_ref[...], preferred_element_type=jnp.float32)
    dinv = dinv_ref[...]
    h = jnp.maximum(acc * dinv + b_ref[...], 0.0) * dinv
    o_ref[0] = h[:, :D_H // 2]
    o_ref[1] = h[:, D_H // 2:]


def _mm1(p0, w0, dinv2d, b0):
    return pl.pallas_call(
        _mm1_body,
        out_shape=jax.ShapeDtypeStruct((NC, NP, D_H // 2), jnp.float32),
        grid=(NP // BN,),
        in_specs=[
            pl.BlockSpec((NC, BN, D_IN), lambda i: (0, i, 0)),
            pl.BlockSpec((D_IN, D_H), lambda i: (0, 0)),
            pl.BlockSpec((BN, 1), lambda i: (i, 0)),
            pl.BlockSpec((1, D_H), lambda i: (0, 0)),
        ],
        out_specs=pl.BlockSpec((NC, BN, D_H // 2), lambda i: (0, i, 0)),
    )(p0, w0, dinv2d, b0)


def _mm2_body(p_ref, w_ref, dinv_ref, b_ref, o_ref):
    acc = jnp.dot(p_ref[0], w_ref[0], preferred_element_type=jnp.float32)
    acc += jnp.dot(p_ref[1], w_ref[1], preferred_element_type=jnp.float32)
    o_ref[...] = jnp.maximum(acc * dinv_ref[...] + b_ref[...], 0.0)


def _mm2(p1, w1r, dinv2d, b1):
    return pl.pallas_call(
        _mm2_body,
        out_shape=jax.ShapeDtypeStruct((NP, D_OUT), jnp.float32),
        grid=(NP // BN,),
        in_specs=[
            pl.BlockSpec((NC, BN, D_H // 2), lambda i: (0, i, 0)),
            pl.BlockSpec((NC, D_H // 2, D_OUT), lambda i: (0, 0, 0)),
            pl.BlockSpec((BN, 1), lambda i: (i, 0)),
            pl.BlockSpec((1, D_OUT), lambda i: (0, 0)),
        ],
        out_specs=pl.BlockSpec((BN, D_OUT), lambda i: (i, 0)),
    )(p1, w1r, dinv2d, b1)


def kernel(x, edge_index, W0, b0, W1, b1):
    src = edge_index[0]
    dst = edge_index[1]
    pad = EP - E
    srcp = jnp.concatenate([src, jnp.zeros((pad,), jnp.int32)])
    dstp = jnp.concatenate([dst, jnp.full((pad,), N, jnp.int32)])

    src_rows = srcp.reshape(ROWS, 128)
    dst_rows = dstp.reshape(ROWS, 128)

    deg_parts = _deg_kernel(dst_rows)
    deg = deg_parts[0, :N, 0] + deg_parts[1, :N, 0] + 1.0
    dinv = lax.rsqrt(deg)
    dinv2d = jnp.pad(dinv[:, None], ((0, NP - N), (0, 0)))

    xs = jnp.pad(x * dinv2d[:N], ((0, NP - N), (0, 0)))
    q0 = _prop_es(src_rows, dst_rows, xs)

    hs = _mm1(q0.reshape(NC, NP, D_IN), W0, dinv2d, b0.reshape(1, D_H))

    q1 = _prop_cs(src_rows, dst_rows, hs.reshape(NC * NP, D_H // 2))

    out = _mm2(q1.reshape(NC, NP, D_H // 2), W1.reshape(NC, D_H // 2, D_OUT),
               dinv2d, b1.reshape(1, D_OUT))
    return out[:N]


# speedup vs baseline: 11.3822x; 1.0728x over previous
import functools

import jax
import jax.numpy as jnp
from jax import lax
from jax.experimental import pallas as pl
from jax.experimental.pallas import tpu as pltpu
from jax.experimental.pallas import tpu_sc as plsc

N = 10000
E = 320000
D_IN = 128
D_H = 256
D_OUT = 1024

NC = 2
NS = 16
LANES = 16
NW = NC * NS

NP = 10240
EP = 327680
ROWS = EP // 128
RPS = ROWS // NS
RPW = ROWS // NW
NPS = NP // NS

_mesh = plsc.VectorSubcoreMesh(core_axis_name="c", subcore_axis_name="s")


@functools.partial(
    pl.kernel,
    out_type=jax.ShapeDtypeStruct((NC, NP, LANES), jnp.float32),
    mesh=_mesh,
    scratch_types=[
        pltpu.VMEM((RPW, 128), jnp.int32),
        pltpu.VMEM((128, LANES), jnp.float32),
        pltpu.VMEM((128, LANES), jnp.float32),
        pltpu.VMEM_SHARED((NP, LANES), jnp.float32),
    ],
)
def _deg_kernel(dst_hbm, deg_hbm, dst_v, ones_v, zeros_v, hist_sp):
    c = lax.axis_index("c")
    s = lax.axis_index("s")
    w = c * NS + s

    pltpu.sync_copy(dst_hbm.at[pl.ds(w * RPW, RPW)], dst_v)

    one = jnp.full((LANES,), 1.0, jnp.float32)
    zero = jnp.zeros((LANES,), jnp.float32)

    @pl.loop(0, 128)
    def _(i):
        ones_v[i] = one
        zeros_v[i] = zero

    @pl.loop(0, NPS // 128)
    def _(k):
        pltpu.sync_copy(zeros_v, hist_sp.at[pl.ds(s * NPS + k * 128, 128)])

    plsc.subcore_barrier()

    @pl.loop(0, RPW)
    def _(t):
        pltpu.sync_copy(ones_v, hist_sp.at[dst_v.at[t]], add=True)

    plsc.subcore_barrier()
    pltpu.sync_copy(
        hist_sp.at[pl.ds(s * NPS, NPS)],
        deg_hbm.at[c].at[pl.ds(s * NPS, NPS)],
    )


def _edge_chunk(xs_hbm, acc_sp, src_v, dst_v, data0, data1, sem, ch):
    pltpu.sync_copy(xs_hbm.at[src_v.at[0]], data0)

    @pl.loop(0, ch // 2)
    def _(k):
        t = 2 * k
        nxt = jnp.minimum(t + 2, ch - 1)
        cp = pltpu.async_copy(xs_hbm.at[src_v.at[t + 1]], data1, sem)
        pltpu.sync_copy(data0, acc_sp.at[dst_v.at[t]], add=True)
        cp.wait()
        cp2 = pltpu.async_copy(xs_hbm.at[src_v.at[nxt]], data0, sem)
        pltpu.sync_copy(data1, acc_sp.at[dst_v.at[t + 1]], add=True)
        cp2.wait()


CH = 40


@functools.partial(
    pl.kernel,
    out_type=jax.ShapeDtypeStruct((NC * NP, D_IN), jnp.float32),
    mesh=_mesh,
    scratch_types=[
        pltpu.VMEM((CH, 128), jnp.int32),
        pltpu.VMEM((CH, 128), jnp.int32),
        pltpu.VMEM((128, D_IN), jnp.float32),
        pltpu.VMEM((128, D_IN), jnp.float32),
        pltpu.SemaphoreType.DMA,
        pltpu.VMEM_SHARED((NP, D_IN), jnp.float32),
    ],
)
def _prop_es(src_hbm, dst_hbm, xs_hbm, out_hbm,
             src_v, dst_v, data0, data1, sem, acc_sp):
    c = lax.axis_index("c")
    s = lax.axis_index("s")
    base = c * (ROWS // NC) + s * RPW

    @pl.when(c == 0)
    def _():
        pltpu.sync_copy(
            xs_hbm.at[pl.ds(s * NPS, NPS)],
            acc_sp.at[pl.ds(s * NPS, NPS)],
        )

    @pl.when(c != 0)
    def _():
        zero = jnp.zeros((LANES,), jnp.float32)

        @pl.loop(0, 128)
        def _(i):
            @pl.loop(0, D_IN, step=LANES)
            def _(j):
                data0[i, pl.ds(j, LANES)] = zero

        @pl.loop(0, NPS // 128)
        def _(k):
            pltpu.sync_copy(data0, acc_sp.at[pl.ds(s * NPS + k * 128, 128)])

    plsc.subcore_barrier()

    for p in range(RPW // CH):
        pltpu.sync_copy(src_hbm.at[pl.ds(base + p * CH, CH)], src_v)
        pltpu.sync_copy(dst_hbm.at[pl.ds(base + p * CH, CH)], dst_v)
        _edge_chunk(xs_hbm, acc_sp, src_v, dst_v, data0, data1, sem, CH)

    plsc.subcore_barrier()
    pltpu.sync_copy(
        acc_sp.at[pl.ds(s * NPS, NPS)],
        out_hbm.at[pl.ds(c * NP + s * NPS, NPS)],
    )


@functools.partial(
    pl.kernel,
    out_type=jax.ShapeDtypeStruct((NC * NP, D_H // 2), jnp.float32),
    mesh=_mesh,
    scratch_types=[
        pltpu.VMEM((CH, 128), jnp.int32),
        pltpu.VMEM((CH, 128), jnp.int32),
        pltpu.VMEM((128, D_H // 2), jnp.float32),
        pltpu.VMEM((128, D_H // 2), jnp.float32),
        pltpu.SemaphoreType.DMA,
        pltpu.VMEM_SHARED((NP, D_H // 2), jnp.float32),
    ],
)
def _prop_cs(src_hbm, dst_hbm, xs_hbm, out_hbm,
             src_v, dst_v, data0, data1, sem, acc_sp):
    c = lax.axis_index("c")
    s = lax.axis_index("s")

    pltpu.sync_copy(
        xs_hbm.at[pl.ds(c * NP + s * NPS, NPS)],
        acc_sp.at[pl.ds(s * NPS, NPS)],
    )
    plsc.subcore_barrier()

    for p in range(RPS // CH):
        pltpu.sync_copy(
            src_hbm.at[c].at[pl.ds(s * RPS + p * CH, CH)], src_v)
        pltpu.sync_copy(dst_hbm.at[pl.ds(s * RPS + p * CH, CH)], dst_v)
        _edge_chunk(xs_hbm, acc_sp, src_v, dst_v, data0, data1, sem, CH)

    plsc.subcore_barrier()
    pltpu.sync_copy(
        acc_sp.at[pl.ds(s * NPS, NPS)],
        out_hbm.at[pl.ds(c * NP + s * NPS, NPS)],
    )


BN = 512


def _mm1_body(p_ref, w_ref, dinv_ref, b_ref, o_ref):
    q = p_ref[0] + p_ref[1]
    acc = jnp.dot(q, w_ref[...], preferred_element_type=jnp.float32)
    dinv = dinv_ref[...]
    h = jnp.maximum(acc * dinv + b_ref[...], 0.0) * dinv
    o_ref[0] = h[:, :D_H // 2]
    o_ref[1] = h[:, D_H // 2:]


def _mm1(p0, w0, dinv2d, b0):
    return pl.pallas_call(
        _mm1_body,
        out_shape=jax.ShapeDtypeStruct((NC, NP, D_H // 2), jnp.float32),
        grid=(NP // BN,),
        in_specs=[
            pl.BlockSpec((NC, BN, D_IN), lambda i: (0, i, 0)),
            pl.BlockSpec((D_IN, D_H), lambda i: (0, 0)),
            pl.BlockSpec((BN, 1), lambda i: (i, 0)),
            pl.BlockSpec((1, D_H), lambda i: (0, 0)),
        ],
        out_specs=pl.BlockSpec((NC, BN, D_H // 2), lambda i: (0, i, 0)),
    )(p0, w0, dinv2d, b0)


def _mm2_body(p_ref, w_ref, dinv_ref, b_ref, o_ref):
    acc = jnp.dot(p_ref[0], w_ref[0], preferred_element_type=jnp.float32)
    acc += jnp.dot(p_ref[1], w_ref[1], preferred_element_type=jnp.float32)
    o_ref[...] = jnp.maximum(acc * dinv_ref[...] + b_ref[...], 0.0)


def _mm2(p1, w1r, dinv2d, b1):
    return pl.pallas_call(
        _mm2_body,
        out_shape=jax.ShapeDtypeStruct((NP, D_OUT), jnp.float32),
        grid=(NP // BN,),
        in_specs=[
            pl.BlockSpec((NC, BN, D_H // 2), lambda i: (0, i, 0)),
            pl.BlockSpec((NC, D_H // 2, D_OUT), lambda i: (0, 0, 0)),
            pl.BlockSpec((BN, 1), lambda i: (i, 0)),
            pl.BlockSpec((1, D_OUT), lambda i: (0, 0)),
        ],
        out_specs=pl.BlockSpec((BN, D_OUT), lambda i: (i, 0)),
    )(p1, w1r, dinv2d, b1)


def kernel(x, edge_index, W0, b0, W1, b1):
    src = edge_index[0]
    dst = edge_index[1]
    pad = EP - E
    srcp = jnp.concatenate([src, jnp.zeros((pad,), jnp.int32)])
    dstp = jnp.concatenate([dst, jnp.full((pad,), N, jnp.int32)])

    src_rows = srcp.reshape(ROWS, 128)
    dst_rows = dstp.reshape(ROWS, 128)
    src_rows2 = jnp.stack([srcp, srcp + NP]).reshape(NC, ROWS, 128)

    deg_parts = _deg_kernel(dst_rows)
    deg = deg_parts[0, :N, 0] + deg_parts[1, :N, 0] + 1.0
    dinv = lax.rsqrt(deg)
    dinv2d = jnp.pad(dinv[:, None], ((0, NP - N), (0, 0)))

    xs = jnp.pad(x * dinv2d[:N], ((0, NP - N), (0, 0)))
    q0 = _prop_es(src_rows, dst_rows, xs)

    hs = _mm1(q0.reshape(NC, NP, D_IN), W0, dinv2d, b0.reshape(1, D_H))

    q1 = _prop_cs(src_rows2, dst_rows, hs.reshape(NC * NP, D_H // 2))

    out = _mm2(q1.reshape(NC, NP, D_H // 2), W1.reshape(NC, D_H // 2, D_OUT),
               dinv2d, b1.reshape(1, D_OUT))
    return out[:N]
